# baseline (speedup 1.0000x reference)
import jax
import jax.numpy as jnp
from jax import lax
from jax.experimental import pallas as pl
from jax.experimental.pallas import tpu as pltpu
from jax.experimental.pallas import tpu_sc as plsc

B, S, L = 4, 8192, 4096
NC, NSUB = 2, 16
NW = NC * NSUB
CW = L // NW
G = CW // 16
R = 256
NCH = S // R


def _sc_body(x_hbm, o_hbm, buf0, buf1, ob, sem0, sem1):
    wid = lax.axis_index("c") * NSUB + lax.axis_index("s")
    c0 = wid * CW

    def copy_in(b, ch, buf, sem):
        return pltpu.make_async_copy(
            x_hbm.at[b, pl.ds(ch * R, R), pl.ds(c0, CW)], buf, sem)

    def rowloop(buf, base, carry):
        def row_body(r, cr):
            mins, idxs = cr
            rvec = jnp.full((16,), base + r, dtype=jnp.int32)
            nm, ni = [], []
            for g in range(G):
                v = buf[r, pl.ds(g * 16, 16)]
                m = v < mins[g]
                nm.append(jnp.where(m, v, mins[g]))
                ni.append(jnp.where(m, rvec, idxs[g]))
            return (tuple(nm), tuple(ni))
        return lax.fori_loop(0, R, row_body, carry, unroll=2)

    for b in range(B):
        copy_in(b, 0, buf0, sem0).start()
        copy_in(b, 1, buf1, sem1).start()
        init = (
            tuple(jnp.full((16,), jnp.inf, jnp.float32) for _ in range(G)),
            tuple(jnp.zeros((16,), jnp.int32) for _ in range(G)),
        )

        def pair_body(p, carry, b=b):
            copy_in(b, 2 * p, buf0, sem0).wait()
            carry = rowloop(buf0, 2 * p * R, carry)

            @pl.when(p + 1 < NCH // 2)
            def _():
                copy_in(b, 2 * p + 2, buf0, sem0).start()

            copy_in(b, 2 * p + 1, buf1, sem1).wait()
            carry = rowloop(buf1, (2 * p + 1) * R, carry)

            @pl.when(p + 1 < NCH // 2)
            def _():
                copy_in(b, 2 * p + 3, buf1, sem1).start()

            return carry

        _, idxs = lax.fori_loop(0, NCH // 2, pair_body, init)
        for g in range(G):
            ob[pl.ds(g * 16, 16)] = idxs[g]
        pltpu.sync_copy(ob, o_hbm.at[b, pl.ds(c0, CW)])


def kernel(x):
    mesh = plsc.VectorSubcoreMesh(core_axis_name="c", subcore_axis_name="s")
    return pl.kernel(
        _sc_body,
        out_type=jax.ShapeDtypeStruct((B, L), jnp.int32),
        mesh=mesh,
        scratch_types=[
            pltpu.VMEM((R, CW), jnp.float32),
            pltpu.VMEM((R, CW), jnp.float32),
            pltpu.VMEM((CW,), jnp.int32),
            pltpu.SemaphoreType.DMA,
            pltpu.SemaphoreType.DMA,
        ],
    )(x)
